# baseline (speedup 1.0000x reference)
import functools

import jax
import jax.numpy as jnp
from jax import lax
from jax.experimental import pallas as pl
from jax.experimental.pallas import tpu as pltpu
from jax.experimental.pallas import tpu_sc as plsc

NC = 2
NS = 16
NW = NC * NS
LANES = 16
GSUB = 128


def _sc_embed(idx_flat, tok_flat, pos_flat, B, T, V, D):
    b_per_w = B // NW
    nbblk = b_per_w // 128
    ngath = b_per_w // GSUB
    groups16 = b_per_w // LANES
    tv = T * V
    rows_per_tile = tv // NS
    t_per_tile = rows_per_tile // V
    mesh = plsc.VectorSubcoreMesh(
        core_axis_name="c", subcore_axis_name="s", num_cores=NC, num_subcores=NS
    )

    @functools.partial(
        pl.kernel,
        out_type=(
            jax.ShapeDtypeStruct((T, D // 8, B // 128, 8, 128), jnp.float32),
            jax.ShapeDtypeStruct((NC * tv, D), jnp.float32),
        ),
        mesh=mesh,
        compiler_params=pltpu.CompilerParams(
            needs_layout_passes=False, use_tc_tiling_on_sc=False
        ),
        scratch_types=[
            pltpu.VMEM((V * D,), jnp.float32),
            pltpu.VMEM((T * D,), jnp.float32),
            pltpu.VMEM((rows_per_tile, D), jnp.float32),
            pltpu.VMEM((B // NW * T,), jnp.int32),
            pltpu.VMEM((B // NW,), jnp.int32),
            pltpu.VMEM((B // NW, D), jnp.float32),
            pltpu.VMEM((D // 8, B // NW // 128, 8, 128), jnp.float32),
            pltpu.SemaphoreType.DMA,
        ],
    )
    def k(idx_hbm, tok_hbm, pos_hbm, out_hbm, fused_hbm,
          tok_v, pos_v, build_v, idxb_v, fidx_v, rows_v, trans_v, sem):
        c = lax.axis_index("c")
        s = lax.axis_index("s")
        wid = s * NC + c
        pltpu.sync_copy(tok_hbm, tok_v)
        pltpu.sync_copy(pos_hbm, pos_v)
        iota = lax.iota(jnp.int32, LANES)
        iotaT = iota * T
        ihig = iota // 8
        ilow = iota % 8

        for dd in range(D // LANES):
            pos_chunks = [
                pos_v[pl.ds((s * t_per_tile + tt) * D + dd * LANES, LANES)]
                for tt in range(t_per_tile)
            ]
            for v in range(V):
                tokc = tok_v[pl.ds(v * D + dd * LANES, LANES)]
                for tt in range(t_per_tile):
                    build_v[tt * V + v, pl.ds(dd * LANES, LANES)] = (
                        tokc + pos_chunks[tt]
                    )
        pltpu.sync_copy(
            build_v,
            fused_hbm.at[pl.ds(c * tv + s * rows_per_tile, rows_per_tile)],
        )
        plsc.subcore_barrier()

        pltpu.sync_copy(idx_hbm.at[pl.ds(wid * b_per_w * T, b_per_w * T)],
                        idxb_v)
        cbase = c * tv

        def t_body(t, carry):
            tbase = t * V + cbase
            for g in range(groups16):
                vidx = plsc.load_gather(
                    idxb_v, [iotaT + (g * LANES * T + t)]
                )
                fidx_v[pl.ds(g * LANES, LANES)] = vidx + tbase
            copies = [
                pltpu.async_copy(
                    fused_hbm.at[fidx_v.at[pl.ds(j * GSUB, GSUB)]],
                    rows_v.at[pl.ds(j * GSUB, GSUB)],
                    sem,
                )
                for j in range(ngath)
            ]
            for cp in copies:
                cp.wait()

            @plsc.parallel_loop(0, b_per_w)
            def transpose_body(bi):
                bb = bi // 128
                b1 = bi % 128
                for dc in range(D // LANES):
                    val = rows_v[bi, pl.ds(dc * LANES, LANES)]
                    plsc.store_scatter(
                        trans_v,
                        [jnp.full((LANES,), dc * 2, jnp.int32) + ihig,
                         jnp.full((LANES,), bb, jnp.int32),
                         ilow,
                         jnp.full((LANES,), b1, jnp.int32)],
                        val,
                    )

            for dblk in range(D // 8):
                pltpu.sync_copy(
                    trans_v.at[dblk],
                    out_hbm.at[t, dblk, pl.ds(wid * nbblk, nbblk)],
                )
            return carry

        lax.fori_loop(0, T, t_body, 0)

    out5d, _ = k(idx_flat, tok_flat, pos_flat)
    return out5d


def kernel(idx, tok_emb, pos_emb):
    B, T = idx.shape
    V, D = tok_emb.shape
    out5d = _sc_embed(
        idx.reshape(-1),
        tok_emb.reshape(-1),
        pos_emb.reshape(-1),
        B, T, V, D,
    )
    return out5d.transpose(2, 4, 0, 1, 3).reshape(B, T, D)


# speedup vs baseline: 1.0703x; 1.0703x over previous
import functools

import jax
import jax.numpy as jnp
from jax import lax
from jax.experimental import pallas as pl
from jax.experimental.pallas import tpu as pltpu
from jax.experimental.pallas import tpu_sc as plsc

NC = 2
NS = 16
NW = NC * NS
LANES = 16
GSUB = 128


def _sc_embed(idx_flat, tok_flat, pos_flat, B, T, V, D):
    b_per_w = B // NW
    nbblk = b_per_w // 128
    ngath = b_per_w // GSUB
    groups16 = b_per_w // LANES
    tv = T * V
    rows_per_tile = tv // NS
    t_per_tile = rows_per_tile // V
    mesh = plsc.VectorSubcoreMesh(
        core_axis_name="c", subcore_axis_name="s", num_cores=NC, num_subcores=NS
    )

    @functools.partial(
        pl.kernel,
        out_type=(
            jax.ShapeDtypeStruct((T, D // 8, B // 128, 8, 128), jnp.float32),
            jax.ShapeDtypeStruct((NC * tv, D), jnp.float32),
        ),
        mesh=mesh,
        compiler_params=pltpu.CompilerParams(
            needs_layout_passes=False, use_tc_tiling_on_sc=False
        ),
        scratch_types=[
            pltpu.VMEM((V * D,), jnp.float32),
            pltpu.VMEM((T * D,), jnp.float32),
            pltpu.VMEM((rows_per_tile, D), jnp.float32),
            pltpu.VMEM((B // NW * T,), jnp.int32),
            pltpu.VMEM((B // NW,), jnp.int32),
            pltpu.VMEM((B // NW, D), jnp.float32),
            pltpu.VMEM((D // 8, B // NW // 128, 8, 128), jnp.float32),
            pltpu.SemaphoreType.DMA,
        ],
    )
    def k(idx_hbm, tok_hbm, pos_hbm, out_hbm, fused_hbm,
          tok_v, pos_v, build_v, idxb_v, fidx_v, rows_v, trans_v, sem):
        c = lax.axis_index("c")
        s = lax.axis_index("s")
        wid = s * NC + c
        pltpu.sync_copy(tok_hbm, tok_v)
        pltpu.sync_copy(pos_hbm, pos_v)
        iota = lax.iota(jnp.int32, LANES)
        iotaT = iota * T
        ihig = iota // 8
        ilow = iota % 8

        for dd in range(D // LANES):
            pos_chunks = [
                pos_v[pl.ds((s * t_per_tile + tt) * D + dd * LANES, LANES)]
                for tt in range(t_per_tile)
            ]
            for v in range(V):
                tokc = tok_v[pl.ds(v * D + dd * LANES, LANES)]
                for tt in range(t_per_tile):
                    build_v[tt * V + v, pl.ds(dd * LANES, LANES)] = (
                        tokc + pos_chunks[tt]
                    )
        pltpu.sync_copy(
            build_v,
            fused_hbm.at[pl.ds(c * tv + s * rows_per_tile, rows_per_tile)],
        )
        plsc.subcore_barrier()

        pltpu.sync_copy(idx_hbm.at[pl.ds(wid * b_per_w * T, b_per_w * T)],
                        idxb_v)
        cbase = c * tv

        def t_body(t, carry):
            tbase = t * V + cbase
            for g in range(groups16):
                vidx = plsc.load_gather(
                    idxb_v, [iotaT + (g * LANES * T + t)]
                )
                fidx_v[pl.ds(g * LANES, LANES)] = vidx + tbase
            copies = [
                pltpu.async_copy(
                    fused_hbm.at[fidx_v.at[pl.ds(j * GSUB, GSUB)]],
                    rows_v.at[pl.ds(j * GSUB, GSUB)],
                    sem,
                )
                for j in range(ngath)
            ]
            for cp in copies:
                cp.wait()

            @plsc.parallel_loop(0, D)
            def transpose_body(d):
                dblk = d // 8
                d8 = d % 8
                for bb in range(nbblk):
                    for g in range(128 // LANES):
                        rvec = plsc.load_gather(
                            rows_v,
                            [jnp.full((LANES,), bb * 128 + g * LANES,
                                      jnp.int32) + iota,
                             jnp.full((LANES,), d, jnp.int32)],
                        )
                        trans_v[dblk, bb, d8, pl.ds(g * LANES, LANES)] = rvec

            for dblk in range(D // 8):
                pltpu.sync_copy(
                    trans_v.at[dblk],
                    out_hbm.at[t, dblk, pl.ds(wid * nbblk, nbblk)],
                )
            return carry

        lax.fori_loop(0, T, t_body, 0)

    out5d, _ = k(idx_flat, tok_flat, pos_flat)
    return out5d


def kernel(idx, tok_emb, pos_emb):
    B, T = idx.shape
    V, D = tok_emb.shape
    out5d = _sc_embed(
        idx.reshape(-1),
        tok_emb.reshape(-1),
        pos_emb.reshape(-1),
        B, T, V, D,
    )
    return out5d.transpose(2, 4, 0, 1, 3).reshape(B, T, D)


# speedup vs baseline: 1.1525x; 1.0768x over previous
import functools

import jax
import jax.numpy as jnp
from jax import lax
from jax.experimental import pallas as pl
from jax.experimental.pallas import tpu as pltpu
from jax.experimental.pallas import tpu_sc as plsc

NC = 2
NS = 16
NW = NC * NS
LANES = 16
GSUB = 128


def _sc_embed(idx_flat, tok_flat, pos_flat, B, T, V, D):
    b_per_w = B // NW
    nbblk = b_per_w // 128
    ngath = b_per_w // GSUB
    groups16 = b_per_w // LANES
    tv = T * V
    rows_per_tile = tv // NS
    t_per_tile = rows_per_tile // V
    mesh = plsc.VectorSubcoreMesh(
        core_axis_name="c", subcore_axis_name="s", num_cores=NC, num_subcores=NS
    )

    @functools.partial(
        pl.kernel,
        out_type=(
            jax.ShapeDtypeStruct((T, D // 8, B // 128, 8, 128), jnp.float32),
            jax.ShapeDtypeStruct((NC * tv, D), jnp.float32),
        ),
        mesh=mesh,
        compiler_params=pltpu.CompilerParams(
            needs_layout_passes=False, use_tc_tiling_on_sc=False
        ),
        scratch_types=[
            pltpu.VMEM((V * D,), jnp.float32),
            pltpu.VMEM((T * D,), jnp.float32),
            pltpu.VMEM((rows_per_tile, D), jnp.float32),
            pltpu.VMEM((B // NW * T,), jnp.int32),
            pltpu.VMEM((B // NW,), jnp.int32),
            pltpu.VMEM((B // NW, D), jnp.float32),
            pltpu.VMEM((D // 8, B // NW // 128, 8, 128), jnp.float32),
            pltpu.SemaphoreType.DMA,
            pltpu.SemaphoreType.DMA,
        ],
    )
    def k(idx_hbm, tok_hbm, pos_hbm, out_hbm, fused_hbm,
          tok_v, pos_v, build_v, idxb_v, fidx_v, rows_v, trans_v, sem, sem_o):
        c = lax.axis_index("c")
        s = lax.axis_index("s")
        wid = s * NC + c
        pltpu.sync_copy(tok_hbm, tok_v)
        pltpu.sync_copy(pos_hbm, pos_v)
        iota = lax.iota(jnp.int32, LANES)
        iotaT = iota * T
        ihig = iota // 8
        ilow = iota % 8

        for dd in range(D // LANES):
            pos_chunks = [
                pos_v[pl.ds((s * t_per_tile + tt) * D + dd * LANES, LANES)]
                for tt in range(t_per_tile)
            ]
            for v in range(V):
                tokc = tok_v[pl.ds(v * D + dd * LANES, LANES)]
                for tt in range(t_per_tile):
                    build_v[tt * V + v, pl.ds(dd * LANES, LANES)] = (
                        tokc + pos_chunks[tt]
                    )
        pltpu.sync_copy(
            build_v,
            fused_hbm.at[pl.ds(c * tv + s * rows_per_tile, rows_per_tile)],
        )
        plsc.subcore_barrier()

        pltpu.sync_copy(idx_hbm.at[pl.ds(wid * b_per_w * T, b_per_w * T)],
                        idxb_v)
        cbase = c * tv

        def t_body(t, carry):
            tbase = t * V + cbase
            for g in range(groups16):
                vidx = plsc.load_gather(
                    idxb_v, [iotaT + (g * LANES * T + t)]
                )
                fidx_v[pl.ds(g * LANES, LANES)] = vidx + tbase
            copies = [
                pltpu.async_copy(
                    fused_hbm.at[fidx_v.at[pl.ds(j * GSUB, GSUB)]],
                    rows_v.at[pl.ds(j * GSUB, GSUB)],
                    sem,
                )
                for j in range(ngath)
            ]

            @pl.when(t > 0)
            def _():
                for dblk in range(D // 8):
                    pltpu.make_async_copy(
                        trans_v.at[dblk],
                        out_hbm.at[t, dblk, pl.ds(wid * nbblk, nbblk)],
                        sem_o,
                    ).wait()

            for cp in copies:
                cp.wait()

            @plsc.parallel_loop(0, D)
            def transpose_body(d):
                dblk = d // 8
                d8 = d % 8
                for bb in range(nbblk):
                    for g in range(128 // LANES):
                        rvec = plsc.load_gather(
                            rows_v,
                            [jnp.full((LANES,), bb * 128 + g * LANES,
                                      jnp.int32) + iota,
                             jnp.full((LANES,), d, jnp.int32)],
                        )
                        trans_v[dblk, bb, d8, pl.ds(g * LANES, LANES)] = rvec

            for dblk in range(D // 8):
                pltpu.async_copy(
                    trans_v.at[dblk],
                    out_hbm.at[t, dblk, pl.ds(wid * nbblk, nbblk)],
                    sem_o,
                )
            return carry

        lax.fori_loop(0, T, t_body, 0)
        for dblk in range(D // 8):
            pltpu.make_async_copy(
                trans_v.at[dblk],
                out_hbm.at[T - 1, dblk, pl.ds(wid * nbblk, nbblk)],
                sem_o,
            ).wait()

    out5d, _ = k(idx_flat, tok_flat, pos_flat)
    return out5d


def kernel(idx, tok_emb, pos_emb):
    B, T = idx.shape
    V, D = tok_emb.shape
    out5d = _sc_embed(
        idx.reshape(-1),
        tok_emb.reshape(-1),
        pos_emb.reshape(-1),
        B, T, V, D,
    )
    return out5d.transpose(2, 4, 0, 1, 3).reshape(B, T, D)
